# baseline (speedup 1.0000x reference)
import functools

import jax
import jax.numpy as jnp
from jax import lax
from jax.experimental import pallas as pl
from jax.experimental.pallas import tpu as pltpu
from jax.experimental.pallas import tpu_sc as plsc

N = 10000
E = 320000
G = 32
L = 16
OUT = 64

NCORE = 2
NTILE = 16
E2 = E + N
K = 128
NBLK = 162
EPT = NBLK * K
E2P = NTILE * EPT
CH = 16


def _sc_edge_body(H, h_hbm, s_hbm, d_hbm, edge_hbm, b_hbm, out_hbm,
                  pkb0_v, pkb1_v, sidx0_v, sidx1_v, didx0_v, didx1_v,
                  didxg0_v, didxg1_v, sbuf0_v, sbuf1_v, dbuf0_v, dbuf1_v,
                  buf0_v, buf1_v,
                  den_v, den5_v, rden_v, iota_v, stage_v, b_v,
                  sem_pk0, sem_pk1, sem_gs0, sem_gs1, sem_gd0, sem_gd1,
                  sem_gh0, sem_gh1, sem_sc0, sem_sc1,
                  sp_out, sp_den):
    pkb_s = (pkb0_v, pkb1_v)
    sidx_s = (sidx0_v, sidx1_v)
    didx_s = (didx0_v, didx1_v)
    didxg_s = (didxg0_v, didxg1_v)
    sbuf_s = (sbuf0_v, sbuf1_v)
    dbuf_s = (dbuf0_v, dbuf1_v)
    buf_s = (buf0_v, buf1_v)
    sem_pk = (sem_pk0, sem_pk1)
    sem_gs = (sem_gs0, sem_gs1)
    sem_gd = (sem_gd0, sem_gd1)
    sem_gh = (sem_gh0, sem_gh1)
    sem_sc = (sem_sc0, sem_sc1)
    c = lax.axis_index("c")
    t = lax.axis_index("s")
    HV = H // 16
    lanes = lax.iota(jnp.int32, 16)
    coff = c * N
    zf = jnp.zeros((16,), jnp.float32)

    pltpu.sync_copy(b_hbm, b_v)

    for i in range(80 // 16):
        iota_v[pl.ds(i * 16, 16)] = lanes + i * 16

    def zero_den(i, _):
        for v in range(8):
            den_v[i, pl.ds(v * 16, 16)] = zf
        return 0
    lax.fori_loop(0, 80, zero_den, 0)
    for r in range(CH):
        for v in range(HV):
            stage_v[r, pl.ds(v * 16, 16)] = zf

    def zero_out(ci, _):
        row0 = t * 640 + ci * CH

        @pl.when(row0 < N)
        def _():
            pltpu.sync_copy(stage_v, sp_out.at[pl.ds(row0, CH)])
        return 0
    lax.fori_loop(0, 640 // CH, zero_out, 0)

    @pl.when(t == 0)
    def _():
        pltpu.sync_copy(den_v, sp_den)

    plsc.subcore_barrier()

    def fire(j, slot, first):
        pk_v = pkb_s[slot]
        si_v = sidx_s[slot]
        di_v = didx_s[slot]
        dg_v = didxg_s[slot]
        pltpu.make_async_copy(edge_hbm.at[c, t, j], pk_v,
                              sem_pk[slot]).wait()

        def unpack(hh, _):
            pk = pk_v[pl.ds(hh * 16, 16)]
            isrc = lax.shift_right_logical(pk, 14)
            idst = lax.bitwise_and(pk, 16383)
            si_v[pl.ds(hh * 16, 16)] = isrc
            di_v[pl.ds(hh * 16, 16)] = idst
            dg_v[pl.ds(hh * 16, 16)] = idst + coff
            return 0
        lax.fori_loop(0, K // 16, unpack, 0)

        @pl.when(j + 2 < NBLK)
        def _():
            pltpu.async_copy(edge_hbm.at[c, t, j + 2], pk_v, sem_pk[slot])

        pltpu.async_copy(s_hbm.at[si_v], sbuf_s[slot], sem_gs[slot])
        pltpu.async_copy(d_hbm.at[dg_v], dbuf_s[slot], sem_gd[slot])
        pltpu.async_copy(h_hbm.at[si_v], buf_s[slot], sem_gh[slot])

    def process(j, slot):
        si_v = sidx_s[slot]
        di_v = didx_s[slot]
        dg_v = didxg_s[slot]
        sb_v = sbuf_s[slot]
        db_v = dbuf_s[slot]
        bf_v = buf_s[slot]
        pltpu.make_async_copy(s_hbm.at[si_v], sb_v, sem_gs[slot]).wait()
        pltpu.make_async_copy(d_hbm.at[dg_v], db_v, sem_gd[slot]).wait()
        pltpu.make_async_copy(h_hbm.at[si_v], bf_v, sem_gh[slot]).wait()

        def scale(hh, _):
            a = sb_v[pl.ds(hh * 16, 16)] + db_v[pl.ds(hh * 16, 16)]
            a = jnp.where(a >= 0, a, 0.2 * a)
            eid = t * EPT + j * K + hh * 16 + lanes
            ex = jnp.where(eid < E2, jnp.exp(a), 0.0)
            idst = di_v[pl.ds(hh * 16, 16)]
            dhi = lax.shift_right_logical(idst, 7)
            plsc.addupdate_scatter(den_v, [dhi, lax.bitwise_and(idst, 127)],
                                   ex)
            for e in range(16):
                exs = ex[e]
                row = hh * 16 + e
                for v in range(HV):
                    bf_v[row, pl.ds(v * 16, 16)] = (
                        bf_v[row, pl.ds(v * 16, 16)] * exs)
            return 0
        lax.fori_loop(0, K // 16, scale, 0)
        return pltpu.async_copy(bf_v, sp_out.at[di_v], sem_sc[slot],
                                add=True)

    pltpu.async_copy(edge_hbm.at[c, t, 0], pkb_s[0], sem_pk[0])
    pltpu.async_copy(edge_hbm.at[c, t, 1], pkb_s[1], sem_pk[1])
    fire(jnp.int32(0), 0, True)
    fire(jnp.int32(1), 1, True)

    def pipe(jj, _):
        j = jj * 2
        dsc0 = process(j, 0)
        dsc1 = process(j + 1, 1)
        dsc0.wait()

        @pl.when(j + 2 < NBLK)
        def _():
            fire(j + 2, 0, False)
        dsc1.wait()

        @pl.when(j + 3 < NBLK)
        def _():
            fire(j + 3, 1, False)
        return 0
    lax.fori_loop(0, NBLK // 2, pipe, 0)


    pltpu.sync_copy(den_v, sp_den.at[iota_v], add=True)
    plsc.subcore_barrier()

    pltpu.sync_copy(sp_den.at[pl.ds(t * 5, 5)], den5_v)

    def red(i, _):
        acc = den5_v[i // 8, pl.ds((i % 8) * 16, 16)]
        rden_v[pl.ds(i * 16, 16)] = 1.0 / (acc + 1e-16)
        return 0
    lax.fori_loop(0, 640 // 16, red, 0)

    def phase_c(ci, _):
        row0 = t * 640 + ci * CH

        @pl.when(row0 < N)
        def _():
            pltpu.sync_copy(sp_out.at[pl.ds(row0, CH)], stage_v)
            rsv = rden_v[pl.ds(ci * CH, CH)]
            for r in range(CH):
                rs = rsv[r]
                for v in range(HV):
                    y = stage_v[r, pl.ds(v * 16, 16)] * rs + b_v[pl.ds(v * 16, 16)]
                    y = jnp.where(y > 0, y, jnp.exp(jnp.minimum(y, 0.0)) - 1.0)
                    stage_v[r, pl.ds(v * 16, 16)] = y
            pltpu.sync_copy(stage_v, out_hbm.at[pl.ds(c * N + row0, CH)])
        return 0
    lax.fori_loop(0, 640 // CH, phase_c, 0)


@functools.partial(jax.jit, static_argnames=("H",))
def _sc_edge(h, s, d, edges, b, H):
    mesh = plsc.VectorSubcoreMesh(core_axis_name="c", subcore_axis_name="s")
    f = pl.kernel(
        functools.partial(_sc_edge_body, H),
        out_type=jax.ShapeDtypeStruct((2 * N, H), jnp.float32),
        mesh=mesh,
        compiler_params=pltpu.CompilerParams(needs_layout_passes=False),
        scratch_types=[
            pltpu.VMEM((K,), jnp.int32),
            pltpu.VMEM((K,), jnp.int32),
            pltpu.VMEM((K,), jnp.int32),
            pltpu.VMEM((K,), jnp.int32),
            pltpu.VMEM((K,), jnp.int32),
            pltpu.VMEM((K,), jnp.int32),
            pltpu.VMEM((K,), jnp.int32),
            pltpu.VMEM((K,), jnp.int32),
            pltpu.VMEM((K,), jnp.float32),
            pltpu.VMEM((K,), jnp.float32),
            pltpu.VMEM((K,), jnp.float32),
            pltpu.VMEM((K,), jnp.float32),
            pltpu.VMEM((K, H), jnp.float32),
            pltpu.VMEM((K, H), jnp.float32),
            pltpu.VMEM((80, 128), jnp.float32),
            pltpu.VMEM((5, 128), jnp.float32),
            pltpu.VMEM((640,), jnp.float32),
            pltpu.VMEM((80,), jnp.int32),
            pltpu.VMEM((CH, H), jnp.float32),
            pltpu.VMEM((H,), jnp.float32),
            pltpu.SemaphoreType.DMA,
            pltpu.SemaphoreType.DMA,
            pltpu.SemaphoreType.DMA,
            pltpu.SemaphoreType.DMA,
            pltpu.SemaphoreType.DMA,
            pltpu.SemaphoreType.DMA,
            pltpu.SemaphoreType.DMA,
            pltpu.SemaphoreType.DMA,
            pltpu.SemaphoreType.DMA,
            pltpu.SemaphoreType.DMA,
            pltpu.VMEM_SHARED((N, H), jnp.float32),
            pltpu.VMEM_SHARED((80, 128), jnp.float32),
        ],
    )
    return f(h, s, d, edges, b)


def _tc_matmul_body(x_ref, w_ref, as_ref, ad_ref, h_ref, s_ref, d_ref):
    h = jnp.dot(x_ref[...], w_ref[...], preferred_element_type=jnp.float32)
    h_ref[...] = h
    s_ref[...] = jnp.dot(h, as_ref[...], preferred_element_type=jnp.float32)
    d_ref[...] = jnp.dot(h, ad_ref[...], preferred_element_type=jnp.float32)


@jax.jit
def _tc_matmul(x, w, a_s, a_d):
    M, Din = x.shape
    H = w.shape[1]
    R = 2000
    grid = M // R
    return pl.pallas_call(
        _tc_matmul_body,
        grid=(grid,),
        in_specs=[
            pl.BlockSpec((R, Din), lambda i: (i, 0)),
            pl.BlockSpec((Din, H), lambda i: (0, 0)),
            pl.BlockSpec((H, 1), lambda i: (0, 0)),
            pl.BlockSpec((H, 1), lambda i: (0, 0)),
        ],
        out_specs=[
            pl.BlockSpec((R, H), lambda i: (i, 0)),
            pl.BlockSpec((R, 1), lambda i: (i, 0)),
            pl.BlockSpec((R, 1), lambda i: (i, 0)),
        ],
        out_shape=[
            jax.ShapeDtypeStruct((M, H), jnp.float32),
            jax.ShapeDtypeStruct((M, 1), jnp.float32),
            jax.ShapeDtypeStruct((M, 1), jnp.float32),
        ],
    )(x, w, a_s.reshape(H, 1), a_d.reshape(H, 1))


def _tc_pool_body(x_ref, batch_ref, xn1_ref, xn2_ref, wlh_ref, wln_ref,
                  bl_ref, y1_ref, y2_ref):
    gid = lax.broadcasted_iota(jnp.int32, (G, N), 0)
    oh = (gid == batch_ref[...]).astype(jnp.float32)
    cnt = jnp.maximum(jnp.sum(oh, axis=1, keepdims=True), 1.0)
    p1 = jnp.dot(oh, x_ref[pl.ds(0, N), :], preferred_element_type=jnp.float32) / cnt
    p2 = jnp.dot(oh, x_ref[pl.ds(N, N), :], preferred_element_type=jnp.float32) / cnt
    wl_p = wlh_ref[...]
    wl_n = wln_ref[...]
    bl = bl_ref[...]
    y1_ref[...] = jnp.dot(p1, wl_p, preferred_element_type=jnp.float32) + \
        jnp.dot(xn1_ref[...], wl_n, preferred_element_type=jnp.float32) + bl
    y2_ref[...] = jnp.dot(p2, wl_p, preferred_element_type=jnp.float32) + \
        jnp.dot(xn2_ref[...], wl_n, preferred_element_type=jnp.float32) + bl


@jax.jit
def _tc_pool(x, batch2d, xn1, xn2, wlin_h, wlin_n, blin):
    return pl.pallas_call(
        _tc_pool_body,
        out_shape=[
            jax.ShapeDtypeStruct((G, OUT), jnp.float32),
            jax.ShapeDtypeStruct((G, OUT), jnp.float32),
        ],
    )(x, batch2d, xn1, xn2, wlin_h, wlin_n, blin.reshape(1, OUT))


def _edge_arrays(edge_index, off):
    loops = jnp.arange(N, dtype=jnp.int32)
    src = jnp.concatenate([edge_index[0], loops]) + off
    dst = jnp.concatenate([edge_index[1], loops])
    packed = jnp.left_shift(src, 14) | dst
    pad = E2P - E2
    return jnp.pad(packed, (0, pad)).reshape(NTILE, NBLK, K)


def kernel(x1, x2, edge_index1, edge_index2, batch, half_y, x_norm2_1,
           x_norm2_2, W1, as1, ad1, b1, W2, as2, ad2, b2, W3, as3, ad3, b3,
           Wlin, blin):
    edges = jnp.stack([_edge_arrays(edge_index1, 0),
                       _edge_arrays(edge_index2, N)])

    H3 = W3.shape[1]
    W3p = jnp.pad(W3, ((0, 0), (0, 128 - H3)))
    as3p = jnp.pad(as3, (0, 128 - H3))
    ad3p = jnp.pad(ad3, (0, 128 - H3))
    b3p = jnp.pad(b3, (0, 128 - H3))
    wlin_h = jnp.pad(Wlin[:H3], ((0, 128 - H3), (0, 0)))
    wlin_n = Wlin[H3:]

    x = jnp.concatenate([x1, x2], axis=0)
    for (W, a_s, a_d, b) in ((W1, as1, ad1, b1), (W2, as2, ad2, b2),
                             (W3p, as3p, ad3p, b3p)):
        h, s, d = _tc_matmul(x, W, a_s, a_d)
        x = _sc_edge(h, s.reshape(2 * N), d.reshape(2 * N), edges,
                     b, H=128)

    y1, y2 = _tc_pool(x, batch.reshape(1, N), x_norm2_1, x_norm2_2,
                      wlin_h, wlin_n, blin)
    return (y1, y2)


# speedup vs baseline: 1.0432x; 1.0432x over previous
import functools

import jax
import jax.numpy as jnp
from jax import lax
from jax.experimental import pallas as pl
from jax.experimental.pallas import tpu as pltpu
from jax.experimental.pallas import tpu_sc as plsc

N = 10000
E = 320000
G = 32
L = 16
OUT = 64

NCORE = 2
NTILE = 16
E2 = E + N
K = 128
NBLK = 162
EPT = NBLK * K
E2P = NTILE * EPT
CH = 16


def _sc_edge_body(H, h_hbm, s_hbm, d_hbm, edge_hbm, b_hbm, out_hbm,
                  pkb0_v, pkb1_v, sidx0_v, sidx1_v, didx0_v, didx1_v,
                  didxg0_v, didxg1_v, sbuf0_v, sbuf1_v, dbuf0_v, dbuf1_v,
                  buf0_v, buf1_v,
                  den_v, den5_v, rden_v, iota_v, stage_v, b_v,
                  sem_pk0, sem_pk1, sem_gs0, sem_gs1, sem_gd0, sem_gd1,
                  sem_gh0, sem_gh1, sem_sc0, sem_sc1,
                  sp_out, sp_den):
    pkb_s = (pkb0_v, pkb1_v)
    sidx_s = (sidx0_v, sidx1_v)
    didx_s = (didx0_v, didx1_v)
    didxg_s = (didxg0_v, didxg1_v)
    sbuf_s = (sbuf0_v, sbuf1_v)
    dbuf_s = (dbuf0_v, dbuf1_v)
    buf_s = (buf0_v, buf1_v)
    sem_pk = (sem_pk0, sem_pk1)
    sem_gs = (sem_gs0, sem_gs1)
    sem_gd = (sem_gd0, sem_gd1)
    sem_gh = (sem_gh0, sem_gh1)
    sem_sc = (sem_sc0, sem_sc1)
    c = lax.axis_index("c")
    t = lax.axis_index("s")
    HV = H // 16
    lanes = lax.iota(jnp.int32, 16)
    coff = c * N
    zf = jnp.zeros((16,), jnp.float32)

    pltpu.sync_copy(b_hbm, b_v)

    for i in range(80 // 16):
        iota_v[pl.ds(i * 16, 16)] = lanes + i * 16

    def zero_den(i, _):
        for v in range(8):
            den_v[i, pl.ds(v * 16, 16)] = zf
        return 0
    lax.fori_loop(0, 80, zero_den, 0)
    for r in range(CH):
        for v in range(HV):
            stage_v[r, pl.ds(v * 16, 16)] = zf

    def zero_out(ci, _):
        row0 = t * 640 + ci * CH

        @pl.when(row0 < N)
        def _():
            pltpu.sync_copy(stage_v, sp_out.at[pl.ds(row0, CH)])
        return 0
    lax.fori_loop(0, 640 // CH, zero_out, 0)

    @pl.when(t == 0)
    def _():
        pltpu.sync_copy(den_v, sp_den)

    plsc.subcore_barrier()

    def fire(j, slot, first):
        pk_v = pkb_s[slot]
        si_v = sidx_s[slot]
        di_v = didx_s[slot]
        dg_v = didxg_s[slot]
        if not first:
            pltpu.make_async_copy(
                buf_s[slot], sp_out.at[di_v], sem_sc[slot]).wait()
        pltpu.make_async_copy(edge_hbm.at[c, t, j], pk_v,
                              sem_pk[slot]).wait()

        def unpack(hh, _):
            pk = pk_v[pl.ds(hh * 16, 16)]
            isrc = lax.shift_right_logical(pk, 14)
            idst = lax.bitwise_and(pk, 16383)
            si_v[pl.ds(hh * 16, 16)] = isrc
            di_v[pl.ds(hh * 16, 16)] = idst
            dg_v[pl.ds(hh * 16, 16)] = idst + coff
            return 0
        lax.fori_loop(0, K // 16, unpack, 0)

        @pl.when(j + 2 < NBLK)
        def _():
            pltpu.async_copy(edge_hbm.at[c, t, j + 2], pk_v, sem_pk[slot])

        pltpu.async_copy(s_hbm.at[si_v], sbuf_s[slot], sem_gs[slot])
        pltpu.async_copy(d_hbm.at[dg_v], dbuf_s[slot], sem_gd[slot])
        pltpu.async_copy(h_hbm.at[si_v], buf_s[slot], sem_gh[slot])

    def process(j, slot):
        si_v = sidx_s[slot]
        di_v = didx_s[slot]
        dg_v = didxg_s[slot]
        sb_v = sbuf_s[slot]
        db_v = dbuf_s[slot]
        bf_v = buf_s[slot]
        pltpu.make_async_copy(s_hbm.at[si_v], sb_v, sem_gs[slot]).wait()
        pltpu.make_async_copy(d_hbm.at[dg_v], db_v, sem_gd[slot]).wait()
        pltpu.make_async_copy(h_hbm.at[si_v], bf_v, sem_gh[slot]).wait()

        def scale(hh, _):
            a = sb_v[pl.ds(hh * 16, 16)] + db_v[pl.ds(hh * 16, 16)]
            a = jnp.where(a >= 0, a, 0.2 * a)
            eid = t * EPT + j * K + hh * 16 + lanes
            ex = jnp.where(eid < E2, jnp.exp(a), 0.0)
            idst = di_v[pl.ds(hh * 16, 16)]
            dhi = lax.shift_right_logical(idst, 7)
            plsc.addupdate_scatter(den_v, [dhi, lax.bitwise_and(idst, 127)],
                                   ex)
            for e in range(16):
                exs = ex[e]
                row = hh * 16 + e
                for v in range(HV):
                    bf_v[row, pl.ds(v * 16, 16)] = (
                        bf_v[row, pl.ds(v * 16, 16)] * exs)
            return 0
        lax.fori_loop(0, K // 16, scale, 0)
        pltpu.async_copy(bf_v, sp_out.at[di_v], sem_sc[slot], add=True)

    pltpu.async_copy(edge_hbm.at[c, t, 0], pkb_s[0], sem_pk[0])
    pltpu.async_copy(edge_hbm.at[c, t, 1], pkb_s[1], sem_pk[1])
    fire(jnp.int32(0), 0, True)
    fire(jnp.int32(1), 1, True)

    def pipe(jj, _):
        j = jj * 2
        process(j, 0)

        @pl.when(j + 2 < NBLK)
        def _():
            fire(j + 2, 0, False)
        process(j + 1, 1)

        @pl.when(j + 3 < NBLK)
        def _():
            fire(j + 3, 1, False)
        return 0
    lax.fori_loop(0, NBLK // 2, pipe, 0)

    pltpu.make_async_copy(buf_s[0], sp_out.at[didx_s[0]], sem_sc[0]).wait()
    pltpu.make_async_copy(buf_s[1], sp_out.at[didx_s[1]], sem_sc[1]).wait()


    pltpu.sync_copy(den_v, sp_den.at[iota_v], add=True)
    plsc.subcore_barrier()

    pltpu.sync_copy(sp_den.at[pl.ds(t * 5, 5)], den5_v)

    def red(i, _):
        acc = den5_v[i // 8, pl.ds((i % 8) * 16, 16)]
        rden_v[pl.ds(i * 16, 16)] = 1.0 / (acc + 1e-16)
        return 0
    lax.fori_loop(0, 640 // 16, red, 0)

    def phase_c(ci, _):
        row0 = t * 640 + ci * CH

        @pl.when(row0 < N)
        def _():
            pltpu.sync_copy(sp_out.at[pl.ds(row0, CH)], stage_v)
            rsv = rden_v[pl.ds(ci * CH, CH)]
            for r in range(CH):
                rs = rsv[r]
                for v in range(HV):
                    y = stage_v[r, pl.ds(v * 16, 16)] * rs + b_v[pl.ds(v * 16, 16)]
                    y = jnp.where(y > 0, y, jnp.exp(jnp.minimum(y, 0.0)) - 1.0)
                    stage_v[r, pl.ds(v * 16, 16)] = y
            pltpu.sync_copy(stage_v, out_hbm.at[pl.ds(c * N + row0, CH)])
        return 0
    lax.fori_loop(0, 640 // CH, phase_c, 0)


@functools.partial(jax.jit, static_argnames=("H",))
def _sc_edge(h, s, d, edges, b, H):
    mesh = plsc.VectorSubcoreMesh(core_axis_name="c", subcore_axis_name="s")
    f = pl.kernel(
        functools.partial(_sc_edge_body, H),
        out_type=jax.ShapeDtypeStruct((2 * N, H), jnp.float32),
        mesh=mesh,
        compiler_params=pltpu.CompilerParams(needs_layout_passes=False),
        scratch_types=[
            pltpu.VMEM((K,), jnp.int32),
            pltpu.VMEM((K,), jnp.int32),
            pltpu.VMEM((K,), jnp.int32),
            pltpu.VMEM((K,), jnp.int32),
            pltpu.VMEM((K,), jnp.int32),
            pltpu.VMEM((K,), jnp.int32),
            pltpu.VMEM((K,), jnp.int32),
            pltpu.VMEM((K,), jnp.int32),
            pltpu.VMEM((K,), jnp.float32),
            pltpu.VMEM((K,), jnp.float32),
            pltpu.VMEM((K,), jnp.float32),
            pltpu.VMEM((K,), jnp.float32),
            pltpu.VMEM((K, H), jnp.float32),
            pltpu.VMEM((K, H), jnp.float32),
            pltpu.VMEM((80, 128), jnp.float32),
            pltpu.VMEM((5, 128), jnp.float32),
            pltpu.VMEM((640,), jnp.float32),
            pltpu.VMEM((80,), jnp.int32),
            pltpu.VMEM((CH, H), jnp.float32),
            pltpu.VMEM((H,), jnp.float32),
            pltpu.SemaphoreType.DMA,
            pltpu.SemaphoreType.DMA,
            pltpu.SemaphoreType.DMA,
            pltpu.SemaphoreType.DMA,
            pltpu.SemaphoreType.DMA,
            pltpu.SemaphoreType.DMA,
            pltpu.SemaphoreType.DMA,
            pltpu.SemaphoreType.DMA,
            pltpu.SemaphoreType.DMA,
            pltpu.SemaphoreType.DMA,
            pltpu.VMEM_SHARED((N, H), jnp.float32),
            pltpu.VMEM_SHARED((80, 128), jnp.float32),
        ],
    )
    return f(h, s, d, edges, b)


def _tc_matmul_body(x_ref, w_ref, as_ref, ad_ref, h_ref, s_ref, d_ref):
    h = jnp.dot(x_ref[...], w_ref[...], preferred_element_type=jnp.float32)
    h_ref[...] = h
    s_ref[...] = jnp.dot(h, as_ref[...], preferred_element_type=jnp.float32)
    d_ref[...] = jnp.dot(h, ad_ref[...], preferred_element_type=jnp.float32)


@jax.jit
def _tc_matmul(x, w, a_s, a_d):
    M, Din = x.shape
    H = w.shape[1]
    R = 2000
    grid = M // R
    return pl.pallas_call(
        _tc_matmul_body,
        grid=(grid,),
        in_specs=[
            pl.BlockSpec((R, Din), lambda i: (i, 0)),
            pl.BlockSpec((Din, H), lambda i: (0, 0)),
            pl.BlockSpec((H, 1), lambda i: (0, 0)),
            pl.BlockSpec((H, 1), lambda i: (0, 0)),
        ],
        out_specs=[
            pl.BlockSpec((R, H), lambda i: (i, 0)),
            pl.BlockSpec((R, 1), lambda i: (i, 0)),
            pl.BlockSpec((R, 1), lambda i: (i, 0)),
        ],
        out_shape=[
            jax.ShapeDtypeStruct((M, H), jnp.float32),
            jax.ShapeDtypeStruct((M, 1), jnp.float32),
            jax.ShapeDtypeStruct((M, 1), jnp.float32),
        ],
    )(x, w, a_s.reshape(H, 1), a_d.reshape(H, 1))


def _tc_pool_body(x_ref, batch_ref, xn1_ref, xn2_ref, wlh_ref, wln_ref,
                  bl_ref, y1_ref, y2_ref):
    gid = lax.broadcasted_iota(jnp.int32, (G, N), 0)
    oh = (gid == batch_ref[...]).astype(jnp.float32)
    cnt = jnp.maximum(jnp.sum(oh, axis=1, keepdims=True), 1.0)
    p1 = jnp.dot(oh, x_ref[pl.ds(0, N), :], preferred_element_type=jnp.float32) / cnt
    p2 = jnp.dot(oh, x_ref[pl.ds(N, N), :], preferred_element_type=jnp.float32) / cnt
    wl_p = wlh_ref[...]
    wl_n = wln_ref[...]
    bl = bl_ref[...]
    y1_ref[...] = jnp.dot(p1, wl_p, preferred_element_type=jnp.float32) + \
        jnp.dot(xn1_ref[...], wl_n, preferred_element_type=jnp.float32) + bl
    y2_ref[...] = jnp.dot(p2, wl_p, preferred_element_type=jnp.float32) + \
        jnp.dot(xn2_ref[...], wl_n, preferred_element_type=jnp.float32) + bl


@jax.jit
def _tc_pool(x, batch2d, xn1, xn2, wlin_h, wlin_n, blin):
    return pl.pallas_call(
        _tc_pool_body,
        out_shape=[
            jax.ShapeDtypeStruct((G, OUT), jnp.float32),
            jax.ShapeDtypeStruct((G, OUT), jnp.float32),
        ],
    )(x, batch2d, xn1, xn2, wlin_h, wlin_n, blin.reshape(1, OUT))


def _edge_arrays(edge_index, off):
    loops = jnp.arange(N, dtype=jnp.int32)
    src = jnp.concatenate([edge_index[0], loops]) + off
    dst = jnp.concatenate([edge_index[1], loops])
    packed = jnp.left_shift(src, 14) | dst
    pad = E2P - E2
    return jnp.pad(packed, (0, pad)).reshape(NTILE, NBLK, K)


def kernel(x1, x2, edge_index1, edge_index2, batch, half_y, x_norm2_1,
           x_norm2_2, W1, as1, ad1, b1, W2, as2, ad2, b2, W3, as3, ad3, b3,
           Wlin, blin):
    edges = jnp.stack([_edge_arrays(edge_index1, 0),
                       _edge_arrays(edge_index2, N)])

    H3 = W3.shape[1]
    W3p = jnp.pad(W3, ((0, 0), (0, 128 - H3)))
    as3p = jnp.pad(as3, (0, 128 - H3))
    ad3p = jnp.pad(ad3, (0, 128 - H3))
    b3p = jnp.pad(b3, (0, 128 - H3))
    wlin_h = jnp.pad(Wlin[:H3], ((0, 128 - H3), (0, 0)))
    wlin_n = Wlin[H3:]

    x = jnp.concatenate([x1, x2], axis=0)
    for (W, a_s, a_d, b) in ((W1, as1, ad1, b1), (W2, as2, ad2, b2),
                             (W3p, as3p, ad3p, b3p)):
        h, s, d = _tc_matmul(x, W, a_s, a_d)
        x = _sc_edge(h, s.reshape(2 * N), d.reshape(2 * N), edges,
                     b, H=128)

    y1, y2 = _tc_pool(x, batch.reshape(1, N), x_norm2_1, x_norm2_2,
                      wlin_h, wlin_n, blin)
    return (y1, y2)
